# baseline (speedup 1.0000x reference)
import dataclasses
import functools

import jax
import jax.numpy as jnp
from jax import lax
from jax.experimental import pallas as pl
from jax.experimental.pallas import tpu as pltpu
from jax.experimental.pallas import tpu_sc as plsc

N = 10000
D = 128
NP = 10240
TRASH = NP - N
NSC = 2
NSUB = 16
CHUNK = 128
SHIFT = 14
ROWS_PER_TILE = NP // NSUB
NL = 5
BN_EPS = 1e-5
BN = 1024
G = NP // BN


def _prep_edges(edge_index):
    E = edge_index.shape[1]
    per = NSC * NSUB * CHUNK
    n_chunks = -(-E // per)
    if n_chunks % 2:
        n_chunks += 1
    pad = n_chunks * per - E
    pad_idx = N + (jnp.arange(pad, dtype=jnp.int32) % TRASH)
    src = jnp.concatenate([edge_index[0], pad_idx])
    dst = jnp.concatenate([edge_index[1], pad_idx])
    packed = (src << SHIFT) | dst
    return packed.reshape(NSC, NSUB, n_chunks, CHUNK), n_chunks


def _unpack_chunk(packed_v, j, sbuf, dbuf):
    for q in range(CHUNK // 16):
        sl = pl.ds(q * 16, 16)
        v = packed_v[j, sl]
        sbuf[0, sl] = lax.shift_right_logical(v, SHIFT)
        dbuf[0, sl] = lax.bitwise_and(v, (1 << SHIFT) - 1)


def _sc_degree(packed, n_chunks):
    mesh = plsc.VectorSubcoreMesh(core_axis_name="c", subcore_axis_name="s")
    cp = pltpu.CompilerParams()
    if "needs_layout_passes" in pltpu.CompilerParams.__dataclass_fields__:
        cp = dataclasses.replace(cp, needs_layout_passes=False)

    ec = n_chunks * CHUNK

    @functools.partial(
        pl.kernel, mesh=mesh, compiler_params=cp,
        out_type=jax.ShapeDtypeStruct((2 * NSC * NSUB, NP), jnp.float32),
        scratch_types=[
            pltpu.VMEM((ec,), jnp.int32),
            pltpu.VMEM((NP,), jnp.float32),
            pltpu.VMEM((NP,), jnp.float32),
        ],
    )
    def deg_kernel(packed_hbm, out_hbm, packed_v, hist_s, hist_d):
        c = lax.axis_index("c")
        s = lax.axis_index("s")

        @pl.loop(0, NP // 16)
        def _(r):
            zero = jnp.zeros((16,), jnp.float32)
            hist_s[pl.ds(r * 16, 16)] = zero
            hist_d[pl.ds(r * 16, 16)] = zero

        pltpu.sync_copy(packed_hbm.at[c, s], packed_v)
        ones = jnp.full((16,), 1.0, jnp.float32)

        @pl.loop(0, ec // 16)
        def _(j):
            v = packed_v[pl.ds(j * 16, 16)]
            plsc.addupdate_scatter(hist_s, [lax.shift_right_logical(v, SHIFT)], ones)
            plsc.addupdate_scatter(hist_d, [lax.bitwise_and(v, (1 << SHIFT) - 1)], ones)

        w = c * NSUB + s
        pltpu.sync_copy(hist_s, out_hbm.at[w])
        pltpu.sync_copy(hist_d, out_hbm.at[NSC * NSUB + w])

    return deg_kernel(packed.reshape(NSC, NSUB, ec))


def _sc_aggregate(u, packed, n_chunks):
    mesh = plsc.VectorSubcoreMesh(core_axis_name="c", subcore_axis_name="s")

    @functools.partial(
        pl.kernel, mesh=mesh,
        out_type=jax.ShapeDtypeStruct((NSC, NP, D), jnp.float32),
        scratch_types=[
            pltpu.VMEM((n_chunks, CHUNK), jnp.int32),
            pltpu.VMEM((1, CHUNK), jnp.int32),
            pltpu.VMEM((1, CHUNK), jnp.int32),
            pltpu.VMEM((1, CHUNK), jnp.int32),
            pltpu.VMEM((1, CHUNK), jnp.int32),
            pltpu.VMEM((CHUNK, D), jnp.float32),
            pltpu.VMEM((CHUNK, D), jnp.float32),
            pltpu.VMEM_SHARED((NP, D), jnp.float32),
            pltpu.SemaphoreType.DMA,
            pltpu.SemaphoreType.DMA,
            pltpu.SemaphoreType.DMA,
            pltpu.SemaphoreType.DMA,
        ],
    )
    def agg_kernel(u_hbm, packed_hbm, out_hbm, packed_v, sb0, db0, sb1, db1,
                   buf0, buf1, acc, sem0, sem1, sems0, sems1):
        c = lax.axis_index("c")
        s = lax.axis_index("s")
        base = s * ROWS_PER_TILE

        @pl.loop(0, CHUNK)
        def _(r):
            for q in range(D // 16):
                buf0[r, pl.ds(q * 16, 16)] = jnp.zeros((16,), jnp.float32)

        for k in range(ROWS_PER_TILE // CHUNK):
            pltpu.sync_copy(buf0, acc.at[pl.ds(base + k * CHUNK, CHUNK)])
        plsc.subcore_barrier()

        pltpu.sync_copy(packed_hbm.at[c, s], packed_v)

        _unpack_chunk(packed_v, 0, sb0, db0)
        pltpu.make_async_copy(u_hbm.at[sb0.at[0]], buf0, sem0).start()

        @pl.loop(0, n_chunks, step=2)
        def _(j):
            @pl.when(j > 0)
            def _():
                pltpu.make_async_copy(buf1, acc.at[db1.at[0]], sems1).wait()

            _unpack_chunk(packed_v, j + 1, sb1, db1)
            pltpu.make_async_copy(u_hbm.at[sb1.at[0]], buf1, sem1).start()
            pltpu.make_async_copy(u_hbm.at[sb0.at[0]], buf0, sem0).wait()
            pltpu.async_copy(buf0, acc.at[db0.at[0]], sems0, add=True)
            pltpu.make_async_copy(u_hbm.at[sb1.at[0]], buf1, sem1).wait()
            pltpu.async_copy(buf1, acc.at[db1.at[0]], sems1, add=True)

            @pl.when(j + 2 < n_chunks)
            def _():
                pltpu.make_async_copy(buf0, acc.at[db0.at[0]], sems0).wait()
                _unpack_chunk(packed_v, j + 2, sb0, db0)
                pltpu.make_async_copy(u_hbm.at[sb0.at[0]], buf0, sem0).start()

        pltpu.make_async_copy(buf0, acc.at[db0.at[0]], sems0).wait()
        pltpu.make_async_copy(buf1, acc.at[db1.at[0]], sems1).wait()
        plsc.subcore_barrier()
        sl = pl.ds(base, ROWS_PER_TILE)
        pltpu.sync_copy(acc.at[sl], out_hbm.at[c, sl])

    return agg_kernel(u, packed)


def _tc_prep(degs4, h_pad, W1):
    def body(dg_ref, h_ref, w_ref, cs_ref, cd_ref, u_ref):
        d = jnp.sum(dg_ref[...], axis=1)
        dt = jnp.transpose(d)
        ds_ = dt[:, 0:1]
        dd_ = dt[:, 1:2]
        cs = jnp.where(ds_ > 0, lax.rsqrt(jnp.maximum(ds_, 1.0)), 0.0)
        cd = jnp.where(dd_ > 0, lax.rsqrt(jnp.maximum(dd_, 1.0)), 0.0)
        cs_ref[...] = cs
        cd_ref[...] = cd
        u_ref[...] = jnp.dot(h_ref[...] * cs, w_ref[...],
                             preferred_element_type=jnp.float32)

    return pl.pallas_call(
        body,
        grid=(G,),
        in_specs=[pl.BlockSpec((2, NSC * NSUB, BN), lambda i: (0, 0, i)),
                  pl.BlockSpec((BN, D), lambda i: (i, 0)),
                  pl.BlockSpec((D, D), lambda i: (0, 0))],
        out_specs=[pl.BlockSpec((BN, 1), lambda i: (i, 0)),
                   pl.BlockSpec((BN, 1), lambda i: (i, 0)),
                   pl.BlockSpec((BN, D), lambda i: (i, 0))],
        out_shape=[jax.ShapeDtypeStruct((NP, 1), jnp.float32),
                   jax.ShapeDtypeStruct((NP, 1), jnp.float32),
                   jax.ShapeDtypeStruct((NP, D), jnp.float32)],
    )(degs4, h_pad, W1)


def _tc_stats(agg, cd, b):
    def body(a_ref, cd_ref, b_ref, t_ref, st_ref):
        i = pl.program_id(0)
        t = (a_ref[0] + a_ref[1]) * cd_ref[...] + b_ref[...]
        t_ref[...] = t
        rowid = i * BN + lax.broadcasted_iota(jnp.int32, (BN, 1), 0)
        tm = jnp.where(rowid < N, t, 0.0)
        s1 = jnp.sum(tm, axis=0, keepdims=True)
        s2 = jnp.sum(tm * tm, axis=0, keepdims=True)

        @pl.when(i == 0)
        def _():
            st_ref[...] = jnp.zeros((2, D), jnp.float32)

        st_ref[...] += jnp.concatenate([s1, s2], axis=0)

    return pl.pallas_call(
        body,
        grid=(G,),
        in_specs=[pl.BlockSpec((NSC, BN, D), lambda i: (0, i, 0)),
                  pl.BlockSpec((BN, 1), lambda i: (i, 0)),
                  pl.BlockSpec((1, D), lambda i: (0, 0))],
        out_specs=[pl.BlockSpec((BN, D), lambda i: (i, 0)),
                   pl.BlockSpec((2, D), lambda i: (0, 0))],
        out_shape=[jax.ShapeDtypeStruct((NP, D), jnp.float32),
                   jax.ShapeDtypeStruct((2, D), jnp.float32)],
    )(agg, cd, b)


def _tc_matmul(x, cs, W):
    def body(x_ref, cs_ref, w_ref, u_ref):
        u_ref[...] = jnp.dot(x_ref[...] * cs_ref[...], w_ref[...],
                             preferred_element_type=jnp.float32)

    return pl.pallas_call(
        body,
        grid=(G,),
        in_specs=[pl.BlockSpec((BN, D), lambda i: (i, 0)),
                  pl.BlockSpec((BN, 1), lambda i: (i, 0)),
                  pl.BlockSpec((D, D), lambda i: (0, 0))],
        out_specs=pl.BlockSpec((BN, D), lambda i: (i, 0)),
        out_shape=jax.ShapeDtypeStruct((NP, D), jnp.float32),
    )(x, cs, W)


def _tc_norm(t, st, gamma, beta, x):
    def body(t_ref, st_ref, g_ref, be_ref, x_ref, z_ref):
        z_ref[...] = _bn_relu_res(t_ref, st_ref, g_ref, be_ref, x_ref)

    return pl.pallas_call(
        body,
        grid=(G,),
        in_specs=[pl.BlockSpec((BN, D), lambda i: (i, 0)),
                  pl.BlockSpec((2, D), lambda i: (0, 0)),
                  pl.BlockSpec((1, D), lambda i: (0, 0)),
                  pl.BlockSpec((1, D), lambda i: (0, 0)),
                  pl.BlockSpec((BN, D), lambda i: (i, 0))],
        out_specs=pl.BlockSpec((BN, D), lambda i: (i, 0)),
        out_shape=jax.ShapeDtypeStruct((NP, D), jnp.float32),
    )(t, st, gamma, beta, x)


def _bn_relu_res(t_ref, st_ref, g_ref, be_ref, x_ref):
    st = st_ref[...]
    mean = st[0:1] / N
    var = st[1:2] / N - mean * mean
    inv = lax.rsqrt(var + BN_EPS)
    z = (t_ref[...] - mean) * inv * g_ref[...] + be_ref[...]
    return x_ref[...] + jnp.maximum(z, 0.0)


def _tc_norm_mm(t, st, gamma, beta, x, cs, W):
    def body(t_ref, st_ref, g_ref, be_ref, x_ref, cs_ref, w_ref, z_ref,
             u_ref):
        z = _bn_relu_res(t_ref, st_ref, g_ref, be_ref, x_ref)
        z_ref[...] = z
        u_ref[...] = jnp.dot(z * cs_ref[...], w_ref[...],
                             preferred_element_type=jnp.float32)

    return pl.pallas_call(
        body,
        grid=(G,),
        in_specs=[pl.BlockSpec((BN, D), lambda i: (i, 0)),
                  pl.BlockSpec((2, D), lambda i: (0, 0)),
                  pl.BlockSpec((1, D), lambda i: (0, 0)),
                  pl.BlockSpec((1, D), lambda i: (0, 0)),
                  pl.BlockSpec((BN, D), lambda i: (i, 0)),
                  pl.BlockSpec((BN, 1), lambda i: (i, 0)),
                  pl.BlockSpec((D, D), lambda i: (0, 0))],
        out_specs=[pl.BlockSpec((BN, D), lambda i: (i, 0)),
                   pl.BlockSpec((BN, D), lambda i: (i, 0))],
        out_shape=[jax.ShapeDtypeStruct((NP, D), jnp.float32),
                   jax.ShapeDtypeStruct((NP, D), jnp.float32)],
    )(t, st, gamma, beta, x, cs, W)


def _tc_norm_mlp(t, st, gamma, beta, x, M0, b0, M1, b1, M2, b2):
    def body(t_ref, st_ref, g_ref, be_ref, x_ref, m0, c0, m1, c1, m2, c2,
             y_ref):
        y = _bn_relu_res(t_ref, st_ref, g_ref, be_ref, x_ref)
        y = jnp.dot(y, m0[...], preferred_element_type=jnp.float32)
        y = jnp.maximum(y + c0[...], 0.0)
        y = jnp.dot(y, m1[...], preferred_element_type=jnp.float32)
        y = jnp.maximum(y + c1[...], 0.0)
        y = jnp.dot(y, m2[...], preferred_element_type=jnp.float32)
        y_ref[...] = y + c2[...]

    return pl.pallas_call(
        body,
        grid=(G,),
        in_specs=[pl.BlockSpec((BN, D), lambda i: (i, 0)),
                  pl.BlockSpec((2, D), lambda i: (0, 0)),
                  pl.BlockSpec((1, D), lambda i: (0, 0)),
                  pl.BlockSpec((1, D), lambda i: (0, 0)),
                  pl.BlockSpec((BN, D), lambda i: (i, 0)),
                  pl.BlockSpec((D, D // 2), lambda i: (0, 0)),
                  pl.BlockSpec((1, D // 2), lambda i: (0, 0)),
                  pl.BlockSpec((D // 2, D // 4), lambda i: (0, 0)),
                  pl.BlockSpec((1, D // 4), lambda i: (0, 0)),
                  pl.BlockSpec((D // 4, D), lambda i: (0, 0)),
                  pl.BlockSpec((1, D), lambda i: (0, 0))],
        out_specs=pl.BlockSpec((BN, D), lambda i: (i, 0)),
        out_shape=jax.ShapeDtypeStruct((NP, D), jnp.float32),
    )(t, st, gamma, beta, x, M0, b0, M1, b1, M2, b2)


def kernel(h, edge_index, Ws, bs, gammas, betas, M0, mb0, M1, mb1, M2, mb2):
    packed, n_chunks = _prep_edges(edge_index)
    h_pad = jnp.pad(h, ((0, TRASH), (0, 0)))
    degs = _sc_degree(packed, n_chunks)
    cs, cd, u = _tc_prep(degs.reshape(2, NSC * NSUB, NP), h_pad, Ws[0])
    x = h_pad
    for i in range(NL):
        agg = _sc_aggregate(u, packed, n_chunks)
        t, st = _tc_stats(agg, cd, bs[i].reshape(1, D))
        gm = gammas[i].reshape(1, D)
        bt = betas[i].reshape(1, D)
        if i + 1 < NL:
            x, u = _tc_norm_mm(t, st, gm, bt, x, cs, Ws[i + 1])
        else:
            y = _tc_norm_mlp(t, st, gm, bt, x, M0, mb0.reshape(1, -1),
                             M1, mb1.reshape(1, -1), M2, mb2.reshape(1, -1))
    return y[:N]


# speedup vs baseline: 1.2771x; 1.2771x over previous
import dataclasses
import functools

import jax
import jax.numpy as jnp
from jax import lax
from jax.experimental import pallas as pl
from jax.experimental.pallas import tpu as pltpu
from jax.experimental.pallas import tpu_sc as plsc

N = 10000
D = 128
NP = 10240
TRASH = NP - N
NSC = 2
NSUB = 16
CHUNK = 128
SHIFT = 14
ROWS_PER_TILE = NP // NSUB
NL = 5
BN_EPS = 1e-5
BN = 1024
G = NP // BN


def _prep_edges(edge_index):
    E = edge_index.shape[1]
    per = NSC * NSUB * CHUNK
    n_chunks = -(-E // per)
    if n_chunks % 2:
        n_chunks += 1
    pad = n_chunks * per - E
    pad_idx = N + (jnp.arange(pad, dtype=jnp.int32) % TRASH)
    src = jnp.concatenate([edge_index[0], pad_idx])
    dst = jnp.concatenate([edge_index[1], pad_idx])
    packed = (src << SHIFT) | dst
    return packed.reshape(NSC, NSUB, n_chunks, CHUNK), n_chunks


def _unpack_chunk(packed_v, j, sbuf, dbuf):
    for q in range(CHUNK // 16):
        sl = pl.ds(q * 16, 16)
        v = packed_v[j, sl]
        sbuf[0, sl] = lax.shift_right_logical(v, SHIFT)
        dbuf[0, sl] = lax.bitwise_and(v, (1 << SHIFT) - 1)


def _sc_degree(packed, n_chunks):
    mesh = plsc.VectorSubcoreMesh(core_axis_name="c", subcore_axis_name="s")
    cp = pltpu.CompilerParams()
    if "needs_layout_passes" in pltpu.CompilerParams.__dataclass_fields__:
        cp = dataclasses.replace(cp, needs_layout_passes=False)

    ec = n_chunks * CHUNK

    @functools.partial(
        pl.kernel, mesh=mesh, compiler_params=cp,
        out_type=jax.ShapeDtypeStruct((2 * NSC * NSUB, NP), jnp.float32),
        scratch_types=[
            pltpu.VMEM((ec,), jnp.int32),
            pltpu.VMEM((NP,), jnp.float32),
            pltpu.VMEM((NP,), jnp.float32),
        ],
    )
    def deg_kernel(packed_hbm, out_hbm, packed_v, hist_s, hist_d):
        c = lax.axis_index("c")
        s = lax.axis_index("s")

        @pl.loop(0, NP // 16)
        def _(r):
            zero = jnp.zeros((16,), jnp.float32)
            hist_s[pl.ds(r * 16, 16)] = zero
            hist_d[pl.ds(r * 16, 16)] = zero

        pltpu.sync_copy(packed_hbm.at[c, s], packed_v)
        ones = jnp.full((16,), 1.0, jnp.float32)

        @pl.loop(0, ec // 16)
        def _(j):
            v = packed_v[pl.ds(j * 16, 16)]
            plsc.addupdate_scatter(hist_s, [lax.shift_right_logical(v, SHIFT)], ones)
            plsc.addupdate_scatter(hist_d, [lax.bitwise_and(v, (1 << SHIFT) - 1)], ones)

        w = c * NSUB + s
        pltpu.sync_copy(hist_s, out_hbm.at[w])
        pltpu.sync_copy(hist_d, out_hbm.at[NSC * NSUB + w])

    return deg_kernel(packed.reshape(NSC, NSUB, ec))


def _sc_aggregate(u, packed, n_chunks):
    mesh = plsc.VectorSubcoreMesh(core_axis_name="c", subcore_axis_name="s")

    @functools.partial(
        pl.kernel, mesh=mesh,
        out_type=jax.ShapeDtypeStruct((NSC, NP, D), jnp.float32),
        scratch_types=[
            pltpu.VMEM((n_chunks, CHUNK), jnp.int32),
            pltpu.VMEM((1, CHUNK), jnp.int32),
            pltpu.VMEM((1, CHUNK), jnp.int32),
            pltpu.VMEM((1, CHUNK), jnp.int32),
            pltpu.VMEM((1, CHUNK), jnp.int32),
            pltpu.VMEM((CHUNK, D), jnp.float32),
            pltpu.VMEM((CHUNK, D), jnp.float32),
            pltpu.VMEM_SHARED((NP, D), jnp.float32),
            pltpu.SemaphoreType.DMA,
            pltpu.SemaphoreType.DMA,
        ],
    )
    def agg_kernel(u_hbm, packed_hbm, out_hbm, packed_v, sb0, db0, sb1, db1,
                   buf0, buf1, acc, sem0, sem1):
        c = lax.axis_index("c")
        s = lax.axis_index("s")
        base = s * ROWS_PER_TILE

        @pl.loop(0, CHUNK)
        def _(r):
            for q in range(D // 16):
                buf0[r, pl.ds(q * 16, 16)] = jnp.zeros((16,), jnp.float32)

        for k in range(ROWS_PER_TILE // CHUNK):
            pltpu.sync_copy(buf0, acc.at[pl.ds(base + k * CHUNK, CHUNK)])
        plsc.subcore_barrier()

        pltpu.sync_copy(packed_hbm.at[c, s], packed_v)

        _unpack_chunk(packed_v, 0, sb0, db0)
        pltpu.make_async_copy(u_hbm.at[sb0.at[0]], buf0, sem0).start()

        @pl.loop(0, n_chunks, step=2)
        def _(j):
            _unpack_chunk(packed_v, j + 1, sb1, db1)
            pltpu.make_async_copy(u_hbm.at[sb1.at[0]], buf1, sem1).start()
            pltpu.make_async_copy(u_hbm.at[sb0.at[0]], buf0, sem0).wait()
            pltpu.sync_copy(buf0, acc.at[db0.at[0]], add=True)

            @pl.when(j + 2 < n_chunks)
            def _():
                _unpack_chunk(packed_v, j + 2, sb0, db0)
                pltpu.make_async_copy(u_hbm.at[sb0.at[0]], buf0, sem0).start()

            pltpu.make_async_copy(u_hbm.at[sb1.at[0]], buf1, sem1).wait()
            pltpu.sync_copy(buf1, acc.at[db1.at[0]], add=True)

        plsc.subcore_barrier()
        sl = pl.ds(base, ROWS_PER_TILE)
        pltpu.sync_copy(acc.at[sl], out_hbm.at[c, sl])

    return agg_kernel(u, packed)


def _tc_prep(degs4, h_pad, W1):
    def body(dg_ref, h_ref, w_ref, cs_ref, cd_ref, u_ref):
        d = jnp.sum(dg_ref[...], axis=1)
        dt = jnp.transpose(d)
        ds_ = dt[:, 0:1]
        dd_ = dt[:, 1:2]
        cs = jnp.where(ds_ > 0, lax.rsqrt(jnp.maximum(ds_, 1.0)), 0.0)
        cd = jnp.where(dd_ > 0, lax.rsqrt(jnp.maximum(dd_, 1.0)), 0.0)
        cs_ref[...] = cs
        cd_ref[...] = cd
        u_ref[...] = jnp.dot(h_ref[...] * cs, w_ref[...],
                             preferred_element_type=jnp.float32)

    return pl.pallas_call(
        body,
        grid=(G,),
        in_specs=[pl.BlockSpec((2, NSC * NSUB, BN), lambda i: (0, 0, i)),
                  pl.BlockSpec((BN, D), lambda i: (i, 0)),
                  pl.BlockSpec((D, D), lambda i: (0, 0))],
        out_specs=[pl.BlockSpec((BN, 1), lambda i: (i, 0)),
                   pl.BlockSpec((BN, 1), lambda i: (i, 0)),
                   pl.BlockSpec((BN, D), lambda i: (i, 0))],
        out_shape=[jax.ShapeDtypeStruct((NP, 1), jnp.float32),
                   jax.ShapeDtypeStruct((NP, 1), jnp.float32),
                   jax.ShapeDtypeStruct((NP, D), jnp.float32)],
    )(degs4, h_pad, W1)


def _tc_stats(agg, cd, b):
    def body(a_ref, cd_ref, b_ref, t_ref, st_ref):
        i = pl.program_id(0)
        t = (a_ref[0] + a_ref[1]) * cd_ref[...] + b_ref[...]
        t_ref[...] = t
        rowid = i * BN + lax.broadcasted_iota(jnp.int32, (BN, 1), 0)
        tm = jnp.where(rowid < N, t, 0.0)
        s1 = jnp.sum(tm, axis=0, keepdims=True)
        s2 = jnp.sum(tm * tm, axis=0, keepdims=True)

        @pl.when(i == 0)
        def _():
            st_ref[...] = jnp.zeros((2, D), jnp.float32)

        st_ref[...] += jnp.concatenate([s1, s2], axis=0)

    return pl.pallas_call(
        body,
        grid=(G,),
        in_specs=[pl.BlockSpec((NSC, BN, D), lambda i: (0, i, 0)),
                  pl.BlockSpec((BN, 1), lambda i: (i, 0)),
                  pl.BlockSpec((1, D), lambda i: (0, 0))],
        out_specs=[pl.BlockSpec((BN, D), lambda i: (i, 0)),
                   pl.BlockSpec((2, D), lambda i: (0, 0))],
        out_shape=[jax.ShapeDtypeStruct((NP, D), jnp.float32),
                   jax.ShapeDtypeStruct((2, D), jnp.float32)],
    )(agg, cd, b)


def _tc_matmul(x, cs, W):
    def body(x_ref, cs_ref, w_ref, u_ref):
        u_ref[...] = jnp.dot(x_ref[...] * cs_ref[...], w_ref[...],
                             preferred_element_type=jnp.float32)

    return pl.pallas_call(
        body,
        grid=(G,),
        in_specs=[pl.BlockSpec((BN, D), lambda i: (i, 0)),
                  pl.BlockSpec((BN, 1), lambda i: (i, 0)),
                  pl.BlockSpec((D, D), lambda i: (0, 0))],
        out_specs=pl.BlockSpec((BN, D), lambda i: (i, 0)),
        out_shape=jax.ShapeDtypeStruct((NP, D), jnp.float32),
    )(x, cs, W)


def _tc_norm(t, st, gamma, beta, x):
    def body(t_ref, st_ref, g_ref, be_ref, x_ref, z_ref):
        z_ref[...] = _bn_relu_res(t_ref, st_ref, g_ref, be_ref, x_ref)

    return pl.pallas_call(
        body,
        grid=(G,),
        in_specs=[pl.BlockSpec((BN, D), lambda i: (i, 0)),
                  pl.BlockSpec((2, D), lambda i: (0, 0)),
                  pl.BlockSpec((1, D), lambda i: (0, 0)),
                  pl.BlockSpec((1, D), lambda i: (0, 0)),
                  pl.BlockSpec((BN, D), lambda i: (i, 0))],
        out_specs=pl.BlockSpec((BN, D), lambda i: (i, 0)),
        out_shape=jax.ShapeDtypeStruct((NP, D), jnp.float32),
    )(t, st, gamma, beta, x)


def _bn_relu_res(t_ref, st_ref, g_ref, be_ref, x_ref):
    st = st_ref[...]
    mean = st[0:1] / N
    var = st[1:2] / N - mean * mean
    inv = lax.rsqrt(var + BN_EPS)
    z = (t_ref[...] - mean) * inv * g_ref[...] + be_ref[...]
    return x_ref[...] + jnp.maximum(z, 0.0)


def _tc_norm_mm(t, st, gamma, beta, x, cs, W):
    def body(t_ref, st_ref, g_ref, be_ref, x_ref, cs_ref, w_ref, z_ref,
             u_ref):
        z = _bn_relu_res(t_ref, st_ref, g_ref, be_ref, x_ref)
        z_ref[...] = z
        u_ref[...] = jnp.dot(z * cs_ref[...], w_ref[...],
                             preferred_element_type=jnp.float32)

    return pl.pallas_call(
        body,
        grid=(G,),
        in_specs=[pl.BlockSpec((BN, D), lambda i: (i, 0)),
                  pl.BlockSpec((2, D), lambda i: (0, 0)),
                  pl.BlockSpec((1, D), lambda i: (0, 0)),
                  pl.BlockSpec((1, D), lambda i: (0, 0)),
                  pl.BlockSpec((BN, D), lambda i: (i, 0)),
                  pl.BlockSpec((BN, 1), lambda i: (i, 0)),
                  pl.BlockSpec((D, D), lambda i: (0, 0))],
        out_specs=[pl.BlockSpec((BN, D), lambda i: (i, 0)),
                   pl.BlockSpec((BN, D), lambda i: (i, 0))],
        out_shape=[jax.ShapeDtypeStruct((NP, D), jnp.float32),
                   jax.ShapeDtypeStruct((NP, D), jnp.float32)],
    )(t, st, gamma, beta, x, cs, W)


def _tc_norm_mlp(t, st, gamma, beta, x, M0, b0, M1, b1, M2, b2):
    def body(t_ref, st_ref, g_ref, be_ref, x_ref, m0, c0, m1, c1, m2, c2,
             y_ref):
        y = _bn_relu_res(t_ref, st_ref, g_ref, be_ref, x_ref)
        y = jnp.dot(y, m0[...], preferred_element_type=jnp.float32)
        y = jnp.maximum(y + c0[...], 0.0)
        y = jnp.dot(y, m1[...], preferred_element_type=jnp.float32)
        y = jnp.maximum(y + c1[...], 0.0)
        y = jnp.dot(y, m2[...], preferred_element_type=jnp.float32)
        y_ref[...] = y + c2[...]

    return pl.pallas_call(
        body,
        grid=(G,),
        in_specs=[pl.BlockSpec((BN, D), lambda i: (i, 0)),
                  pl.BlockSpec((2, D), lambda i: (0, 0)),
                  pl.BlockSpec((1, D), lambda i: (0, 0)),
                  pl.BlockSpec((1, D), lambda i: (0, 0)),
                  pl.BlockSpec((BN, D), lambda i: (i, 0)),
                  pl.BlockSpec((D, D // 2), lambda i: (0, 0)),
                  pl.BlockSpec((1, D // 2), lambda i: (0, 0)),
                  pl.BlockSpec((D // 2, D // 4), lambda i: (0, 0)),
                  pl.BlockSpec((1, D // 4), lambda i: (0, 0)),
                  pl.BlockSpec((D // 4, D), lambda i: (0, 0)),
                  pl.BlockSpec((1, D), lambda i: (0, 0))],
        out_specs=pl.BlockSpec((BN, D), lambda i: (i, 0)),
        out_shape=jax.ShapeDtypeStruct((NP, D), jnp.float32),
    )(t, st, gamma, beta, x, M0, b0, M1, b1, M2, b2)


def kernel(h, edge_index, Ws, bs, gammas, betas, M0, mb0, M1, mb1, M2, mb2):
    packed, n_chunks = _prep_edges(edge_index)
    h_pad = jnp.pad(h, ((0, TRASH), (0, 0)))
    degs = _sc_degree(packed, n_chunks)
    cs, cd, u = _tc_prep(degs.reshape(2, NSC * NSUB, NP), h_pad, Ws[0])
    x = h_pad
    for i in range(NL):
        agg = _sc_aggregate(u, packed, n_chunks)
        t, st = _tc_stats(agg, cd, bs[i].reshape(1, D))
        gm = gammas[i].reshape(1, D)
        bt = betas[i].reshape(1, D)
        if i + 1 < NL:
            x, u = _tc_norm_mm(t, st, gm, bt, x, cs, Ws[i + 1])
        else:
            y = _tc_norm_mlp(t, st, gm, bt, x, M0, mb0.reshape(1, -1),
                             M1, mb1.reshape(1, -1), M2, mb2.reshape(1, -1))
    return y[:N]
